# baseline (speedup 1.0000x reference)
import functools

import jax
import jax.numpy as jnp
from jax import lax
from jax.experimental import pallas as pl
from jax.experimental.pallas import tpu as pltpu
from jax.experimental.pallas import tpu_sc as plsc

B = 4
S = 2048
D = 768
NC = 2
NS = 16
NW = NC * NS
SPW = S // NW
CH = 32
NSTEP = B * (SPW // CH)
_STEPS = [(b, h) for b in range(B) for h in range(SPW // CH)]

_mesh = plsc.VectorSubcoreMesh(core_axis_name="c", subcore_axis_name="s")


@functools.partial(
    pl.kernel,
    mesh=_mesh,
    out_type=jax.ShapeDtypeStruct((B, S, D), jnp.float32),
    scratch_types=[
        pltpu.VMEM((NSTEP, CH), jnp.int32),
        pltpu.VMEM((SPW, D), jnp.float32),
        pltpu.VMEM((CH, D), jnp.float32),
        pltpu.VMEM((CH, D), jnp.float32),
        pltpu.SemaphoreType.DMA,
        pltpu.SemaphoreType.DMA,
        pltpu.SemaphoreType.DMA,
        pltpu.SemaphoreType.DMA,
    ],
)
def _emb_kernel(x_hbm, tok_hbm, pos_hbm, out_hbm,
                idx_v, pos_v, buf0, buf1, g0, g1, w0, w1):
    wid = lax.axis_index("s") * NC + lax.axis_index("c")
    s0 = wid * SPW

    pltpu.sync_copy(pos_hbm.at[pl.ds(s0, SPW)], pos_v)
    for step, (b, h) in enumerate(_STEPS):
        pltpu.sync_copy(x_hbm.at[b, pl.ds(s0 + h * CH, CH)], idx_v.at[step])

    bufs = [buf0, buf1]
    gsems = [g0, g1]
    wsems = [w0, w1]

    def start_gather(step, p):
        return pltpu.async_copy(tok_hbm.at[idx_v.at[step]], bufs[p], gsems[p])

    def add_pos(p, h):
        buf = bufs[p]

        def body(r, carry):
            for c in range(D // 16):
                sl = pl.ds(c * 16, 16)
                buf[r, sl] = buf[r, sl] + pos_v[h * CH + r, sl]
            return carry

        lax.fori_loop(0, CH, body, 0, unroll=False)

    gcopies = [None, None]
    wcopies = [None, None]
    gcopies[0] = start_gather(0, 0)
    for step in range(NSTEP):
        p = step % 2
        if step + 1 < NSTEP:
            if wcopies[p ^ 1] is not None:
                wcopies[p ^ 1].wait()
            gcopies[p ^ 1] = start_gather(step + 1, p ^ 1)
        gcopies[p].wait()
        b, h = _STEPS[step]
        add_pos(p, h)
        wcopies[p] = pltpu.async_copy(
            bufs[p], out_hbm.at[b, pl.ds(s0 + h * CH, CH)], wsems[p])
    wcopies[0].wait()
    wcopies[1].wait()


def kernel(x, token_table, pos_table):
    return _emb_kernel(x.astype(jnp.int32), token_table, pos_table)
